# baseline (speedup 1.0000x reference)
import jax
import jax.numpy as jnp
from jax.experimental import pallas as pl
from jax.experimental.pallas import tpu as pltpu

N = 32768
H = 768
E = 64
K = 512
CHUNK = 4096
NCHUNK = N // CHUNK
AC = 1024
NA = N // AC
CB = 4096
NB = N // CB
LANE = 128


def _count_ge(pt_ref, midf):
    def cbody(j, acc):
        for k in range(CB // LANE):
            blk = pt_ref[:, pl.ds(j * CB + k * LANE, LANE)]
            acc = acc + (blk >= midf).astype(jnp.int32)
        return acc
    acc = jax.lax.fori_loop(0, NB, cbody, jnp.zeros((E, LANE), jnp.int32))
    return jnp.sum(acc, axis=1, keepdims=True)


def _masked_min_ge(pt_ref, lof):
    def mbody(j, acc):
        for k in range(CB // LANE):
            blk = pt_ref[:, pl.ds(j * CB + k * LANE, LANE)]
            acc = jnp.minimum(acc, jnp.where(blk >= lof, blk, 2.0))
        return acc
    acc = jax.lax.fori_loop(0, NB, mbody, jnp.full((E, LANE), 2.0, jnp.float32))
    return jnp.min(acc, axis=1, keepdims=True)


def _router_body(x_ref, w_ref, logits_ref, probs_ref, rw_ref, ei_ref, pt_ref):
    i = pl.program_id(0)

    xc = x_ref[...]
    w = w_ref[...]

    logits = jax.lax.dot_general(
        xc, w, (((1,), (1,)), ((), ())),
        preferred_element_type=jnp.float32)
    m = jnp.max(logits, axis=1, keepdims=True)
    ex = jnp.exp(logits - m)
    logits_ref[...] = logits
    probs_ref[...] = ex / jnp.sum(ex, axis=1, keepdims=True)

    lt = jax.lax.dot_general(
        w, xc, (((1,), (1,)), ((), ())),
        preferred_element_type=jnp.float32)
    mt = jnp.max(lt, axis=0, keepdims=True)
    ext = jnp.exp(lt - mt)
    pt_ref[:, pl.ds(i * CHUNK, CHUNK)] = ext / jnp.sum(ext, axis=0, keepdims=True)

    @pl.when(i == NCHUNK - 1)
    def _select_and_assign():
        def wcond(carry):
            lo, hi, c_lo, c_hi, r = carry
            done = jnp.logical_or(c_lo == K, hi - lo <= 1)
            return jnp.logical_and(r < 40, jnp.logical_not(jnp.all(done)))

        def wbody(carry):
            lo, hi, c_lo, c_hi, r = carry
            lo_f = jax.lax.bitcast_convert_type(lo, jnp.float32)
            hi_f = jax.lax.bitcast_convert_type(hi, jnp.float32)
            frac = ((c_lo - K).astype(jnp.float32)
                    / jnp.maximum(c_lo - c_hi, 1).astype(jnp.float32))
            mid_itp = jax.lax.bitcast_convert_type(
                lo_f + (hi_f - lo_f) * frac, jnp.int32)
            mid_bis = (lo + hi) // 2
            mid = jnp.where((r % 3) != 2, mid_itp, mid_bis)
            mid = jnp.clip(mid, lo + 1, hi - 1)
            midf = jax.lax.bitcast_convert_type(mid, jnp.float32)
            c_mid = _count_ge(pt_ref, midf)
            ge = c_mid >= K
            return (jnp.where(ge, mid, lo), jnp.where(ge, hi, mid),
                    jnp.where(ge, c_mid, c_lo), jnp.where(ge, c_hi, c_mid),
                    r + 1)

        lo0 = jnp.zeros((E, 1), jnp.int32)
        hi0 = jnp.full((E, 1), 0x3F800001, jnp.int32)
        lo, _, c_lo, _, _ = jax.lax.while_loop(
            wcond, wbody,
            (lo0, hi0, jnp.full((E, 1), N, jnp.int32),
             jnp.zeros((E, 1), jnp.int32), jnp.int32(0)))
        lo_f = jax.lax.bitcast_convert_type(lo, jnp.float32)
        mn = _masked_min_ge(pt_ref, lo_f)
        t = jnp.where(c_lo == K, mn, lo_f)

        eidx = jax.lax.broadcasted_iota(jnp.int32, (E, AC), 0)

        def abody(c, _):
            p = pt_ref[:, pl.ds(c * AC, AC)]
            sel = p >= t
            masked = jnp.where(sel, p, -1.0)
            best = jnp.max(masked, axis=0)
            bi = jnp.min(jnp.where(masked == best[None, :], eidx, E), axis=0)
            fb = jnp.max(p, axis=0)
            fi = jnp.min(jnp.where(p == fb[None, :], eidx, E), axis=0)
            assigned = best >= 0.0
            rw_ref[c, :] = jnp.where(assigned, best, fb)
            ei_ref[c, :] = jnp.where(assigned, bi, fi)
            return 0

        jax.lax.fori_loop(0, NA, abody, 0)


def kernel(x, W):
    b, s, h = x.shape
    xr = x.reshape(N, H)
    logits, probs, rw, ei = pl.pallas_call(
        _router_body,
        grid=(NCHUNK,),
        in_specs=[
            pl.BlockSpec((CHUNK, H), lambda i: (i, 0)),
            pl.BlockSpec((E, H), lambda i: (0, 0)),
        ],
        out_specs=[
            pl.BlockSpec((CHUNK, E), lambda i: (i, 0)),
            pl.BlockSpec((CHUNK, E), lambda i: (i, 0)),
            pl.BlockSpec((NA, AC), lambda i: (0, 0)),
            pl.BlockSpec((NA, AC), lambda i: (0, 0)),
        ],
        out_shape=[
            jax.ShapeDtypeStruct((N, E), jnp.float32),
            jax.ShapeDtypeStruct((N, E), jnp.float32),
            jax.ShapeDtypeStruct((NA, AC), jnp.float32),
            jax.ShapeDtypeStruct((NA, AC), jnp.int32),
        ],
        scratch_shapes=[pltpu.VMEM((E, N), jnp.float32)],
    )(xr, W)
    return rw.reshape(b, s), ei.reshape(b, s), logits, probs
